# baseline (speedup 1.0000x reference)
import jax
import jax.numpy as jnp
from jax import lax
from jax.experimental import pallas as pl
from jax.experimental.pallas import tpu as pltpu
from jax.experimental.pallas import tpu_sc as plsc

_B = 8
_NQ = 8
_L = 2048
_D = 256
_BT = 5
_BS = _B - _BT
_LT = 1024

_NW = 32
_LC = 128
_NU = _BS * _NQ * (_L // _LC) // _NW
_NB = 3


def _tc_body(x_ref, emb_ref, o_ref):
    for qi in range(_NQ):
        o_ref[0, :, qi * _D:(qi + 1) * _D] = x_ref[0, qi] + emb_ref[qi]


def _tc_call(x, quantizer_emb):
    return pl.pallas_call(
        _tc_body,
        grid=(_BT, _L // _LT),
        in_specs=[
            pl.BlockSpec((1, _NQ, _LT, _D), lambda i, j: (i, 0, j, 0)),
            pl.BlockSpec((_NQ, _D), lambda i, j: (0, 0)),
        ],
        out_specs=pl.BlockSpec((1, _LT, _NQ * _D), lambda i, j: (i, j, 0)),
        out_shape=jax.ShapeDtypeStruct((_BT, _L, _NQ * _D), x.dtype),
    )(x, quantizer_emb)


def _sc_body(x_hbm, emb_hbm, out_hbm, emb_v, bufs, ld_sems, st_sems):
    wid = lax.axis_index("s") * 2 + lax.axis_index("c")
    pltpu.sync_copy(emb_hbm, emb_v)

    def unit(cc):
        u = wid + cc * _NW
        b = u // (_NQ * (_L // _LC))
        rem = u - b * (_NQ * (_L // _LC))
        q = rem // (_L // _LC)
        lc = rem - q * (_L // _LC)
        return b, q, lc * _LC

    def start_load(cc, p):
        b, q, l0 = unit(cc)
        pltpu.async_copy(
            x_hbm.at[_BT + b, q, pl.ds(l0, _LC), :],
            bufs.at[p],
            ld_sems.at[p],
        )

    def wait_load(p):
        pltpu.make_async_copy(
            x_hbm.at[0, 0, pl.ds(0, _LC), :], bufs.at[p], ld_sems.at[p]
        ).wait()

    def start_store(cc, p):
        b, q, l0 = unit(cc)
        pltpu.async_copy(
            bufs.at[p],
            out_hbm.at[b, pl.ds(l0, _LC), pl.ds(q * _D, _D)],
            st_sems.at[p],
        )

    def wait_store(p):
        pltpu.make_async_copy(
            bufs.at[p], out_hbm.at[0, pl.ds(0, _LC), pl.ds(0, _D)], st_sems.at[p]
        ).wait()

    start_load(0, 0)

    def round_body(r, carry):
        for par in range(_NB):
            cc = r * _NB + par
            wait_load(par)
            pn = (par + 1) % _NB
            nxt = cc + 1

            @pl.when(cc >= _NB - 1)
            def _():
                wait_store(pn)

            @pl.when(nxt < _NU)
            def _():
                start_load(nxt, pn)

            _, q, _ = unit(cc)
            e = [emb_v[q, pl.ds(j * 16, 16)] for j in range(16)]

            @plsc.parallel_loop(0, _LC, unroll=2)
            def row(l, _p=par, _e=e):
                for j in range(16):
                    sl = pl.ds(j * 16, 16)
                    bufs[_p, l, sl] = bufs[_p, l, sl] + _e[j]

            start_store(cc, par)
        return carry

    lax.fori_loop(0, _NU // _NB, round_body, 0)
    for cc in range(_NU - 2, _NU):
        wait_store(cc % _NB)


def _sc_call(x, quantizer_emb):
    mesh = plsc.VectorSubcoreMesh(core_axis_name="c", subcore_axis_name="s")
    f = pl.kernel(
        _sc_body,
        out_type=jax.ShapeDtypeStruct((_BS, _L, _NQ * _D), jnp.float32),
        mesh=mesh,
        scratch_types=[
            pltpu.VMEM((_NQ, _D), jnp.float32),
            pltpu.VMEM((_NB, _LC, _D), jnp.float32),
            pltpu.SemaphoreType.DMA((_NB,)),
            pltpu.SemaphoreType.DMA((_NB,)),
        ],
    )
    return f(x, quantizer_emb)


def kernel(x, quantizer_emb):
    out_tc = _tc_call(x, quantizer_emb)
    out_sc = _sc_call(x, quantizer_emb)
    return jnp.concatenate([out_tc, out_sc], axis=0)


# speedup vs baseline: 1.4912x; 1.4912x over previous
import jax
import jax.numpy as jnp
from jax import lax
from jax.experimental import pallas as pl
from jax.experimental.pallas import tpu as pltpu
from jax.experimental.pallas import tpu_sc as plsc

_B = 8
_NQ = 8
_L = 2048
_D = 256
_NW = 32
_LPW = _L // (_NW // _B)
_LC = 16
_NCH = _LPW // _LC
_NB = 3
_NR = 10


def _sc_body(x_hbm, emb_hbm, out_hbm, emb_v, bufs, ld_sems, st_sems):
    wid = lax.axis_index("s") * 2 + lax.axis_index("c")
    b = wid // 4
    lbase = (wid - b * 4) * _LPW
    pltpu.sync_copy(emb_hbm, emb_v)

    e = [
        [emb_v[qi, pl.ds(j * 16, 16)] for j in range(16)]
        for qi in range(_NQ)
    ]

    def start_load(cc, p):
        l0 = lbase + cc * _LC
        for qi in range(_NQ):
            pltpu.async_copy(
                x_hbm.at[b, qi, pl.ds(l0, _LC), :],
                bufs.at[p, :, pl.ds(qi * _D, _D)],
                ld_sems.at[p],
            )

    def wait_load(p):
        pltpu.make_async_copy(
            out_hbm.at[0, pl.ds(0, _LC), :], bufs.at[p], ld_sems.at[p]
        ).wait()

    def start_store(cc, p):
        l0 = lbase + cc * _LC
        pltpu.async_copy(
            bufs.at[p], out_hbm.at[b, pl.ds(l0, _LC), :], st_sems.at[p]
        )

    def wait_store(p):
        pltpu.make_async_copy(
            bufs.at[p], out_hbm.at[0, pl.ds(0, _LC), :], st_sems.at[p]
        ).wait()

    def compute(p):
        for qi in range(_NQ):
            @plsc.parallel_loop(0, _LC, unroll=2)
            def row(l, _p=p, _qi=qi):
                for j in range(16):
                    sl = pl.ds(_qi * _D + j * 16, 16)
                    bufs[_p, l, sl] = bufs[_p, l, sl] + e[_qi][j]

    start_load(0, 0)

    def round_body(r, carry):
        for par in range(_NB):
            cc = r * _NB + par
            wait_load(par)
            pn = (par + 1) % _NB

            @pl.when(cc >= _NB - 1)
            def _():
                wait_store(pn)

            start_load(cc + 1, pn)
            compute(par)
            start_store(cc, par)
        return carry

    lax.fori_loop(0, _NR, round_body, 0)

    for cc in range(_NR * _NB, _NCH):
        p = cc % _NB
        wait_load(p)
        pn = (p + 1) % _NB
        wait_store(pn)
        if cc + 1 < _NCH:
            start_load(cc + 1, pn)
        compute(p)
        start_store(cc, p)
    for cc in range(_NCH - 2, _NCH):
        wait_store(cc % _NB)


@jax.jit
def _sc_call(x, quantizer_emb):
    mesh = plsc.VectorSubcoreMesh(core_axis_name="c", subcore_axis_name="s")
    f = pl.kernel(
        _sc_body,
        out_type=jax.ShapeDtypeStruct((_B, _L, _NQ * _D), jnp.float32),
        mesh=mesh,
        scratch_types=[
            pltpu.VMEM((_NQ, _D), jnp.float32),
            pltpu.VMEM((_NB, _LC, _NQ * _D), jnp.float32),
            pltpu.SemaphoreType.DMA((_NB,)),
            pltpu.SemaphoreType.DMA((_NB,)),
        ],
    )
    return f(x, quantizer_emb)


def kernel(x, quantizer_emb):
    return _sc_call(x, quantizer_emb)
